# Initial kernel scaffold; baseline (speedup 1.0000x reference)
import jax
import jax.numpy as jnp
from jax.experimental import pallas as pl


def kernel(x, edge_index, W0, b0, W1, b1, ln0_w, ln0_b, ln1_w, ln1_b):
    raise NotImplementedError("write your pallas kernel here")


# speedup vs baseline: 11.3895x; 11.3895x over previous
import functools

import jax
import jax.numpy as jnp
from jax import lax
from jax.experimental import pallas as pl
from jax.experimental.pallas import tpu as pltpu
from jax.experimental.pallas import tpu_sc as plsc

N = 10000
E = 320000
D = 128
EPS = 1e-5

NC = 2
NS = 16
NW = NC * NS
EW = E // NW
CH = 80
ITERS = EW // CH
RPT = N // NS
DEGW = 16

_mesh = plsc.VectorSubcoreMesh(core_axis_name="c", subcore_axis_name="s")
_sc_params = pltpu.CompilerParams(use_tc_tiling_on_sc=False)


@functools.partial(
    pl.kernel,
    out_type=jax.ShapeDtypeStruct((NC, N, DEGW), jnp.float32),
    mesh=_mesh,
    scratch_types=[
        pltpu.VMEM((CH,), jnp.int32),
        pltpu.VMEM((CH, DEGW), jnp.float32),
        pltpu.VMEM_SHARED((N, DEGW), jnp.float32),
    ],
    compiler_params=_sc_params,
)
def _deg_kernel(dst_hbm, ones_hbm, out_hbm, dst_v, ones_v, acc):
    c = lax.axis_index("c")
    s = lax.axis_index("s")
    wid = s * NC + c
    pltpu.sync_copy(ones_hbm.at[pl.ds(s * RPT, RPT)], acc.at[pl.ds(s * RPT, RPT)])
    pltpu.sync_copy(ones_hbm.at[pl.ds(0, CH)], ones_v)
    plsc.subcore_barrier()

    def body(i, carry):
        base = wid * EW + i * CH
        pltpu.sync_copy(dst_hbm.at[pl.ds(base, CH)], dst_v)
        pltpu.sync_copy(ones_v, acc.at[dst_v], add=True)
        return carry

    lax.fori_loop(0, ITERS, body, 0)
    plsc.subcore_barrier()
    pltpu.sync_copy(acc.at[pl.ds(s * RPT, RPT)], out_hbm.at[c, pl.ds(s * RPT, RPT)])


@functools.partial(
    pl.kernel,
    out_type=jax.ShapeDtypeStruct((NC, N, D), jnp.float32),
    mesh=_mesh,
    scratch_types=[
        pltpu.VMEM((CH,), jnp.int32),
        pltpu.VMEM((CH,), jnp.int32),
        pltpu.VMEM((CH, D), jnp.float32),
        pltpu.VMEM_SHARED((N, D), jnp.float32),
        pltpu.SemaphoreType.DMA,
    ],
    compiler_params=_sc_params,
)
def _conv_kernel(xs_hbm, src_hbm, dst_hbm, out_hbm, src_v, dst_v, rows_v, acc, sem):
    c = lax.axis_index("c")
    s = lax.axis_index("s")
    wid = s * NC + c
    pltpu.sync_copy(xs_hbm.at[pl.ds(s * RPT, RPT)], acc.at[pl.ds(s * RPT, RPT)])
    plsc.subcore_barrier()

    def body(i, carry):
        base = wid * EW + i * CH
        pltpu.sync_copy(src_hbm.at[pl.ds(base, CH)], src_v)
        pltpu.sync_copy(dst_hbm.at[pl.ds(base, CH)], dst_v)
        pltpu.async_copy(xs_hbm.at[src_v], rows_v, sem).wait()
        pltpu.sync_copy(rows_v, acc.at[dst_v], add=True)
        return carry

    lax.fori_loop(0, ITERS, body, 0)
    plsc.subcore_barrier()
    pltpu.sync_copy(acc.at[pl.ds(s * RPT, RPT)], out_hbm.at[c, pl.ds(s * RPT, RPT)])


def _prep_body(p0_ref, p1_ref, x_ref, dinv_ref, xs_ref):
    deg = p0_ref[:, 0:1] + p1_ref[:, 0:1] - 1.0
    dinv = lax.rsqrt(deg)
    dinv_ref[...] = dinv
    xs_ref[...] = x_ref[...] * dinv


_prep = pl.pallas_call(
    _prep_body,
    out_shape=(
        jax.ShapeDtypeStruct((N, 1), jnp.float32),
        jax.ShapeDtypeStruct((N, D), jnp.float32),
    ),
)

MB = 1000
NBLK = N // MB


def _mm_body(residual, *refs):
    if residual:
        p0_ref, p1_ref, xs_ref, dinv_ref, w_ref, b_ref, xres_ref, h_ref, st_ref, acc_ref = refs
    else:
        p0_ref, p1_ref, xs_ref, dinv_ref, w_ref, b_ref, h_ref, st_ref, acc_ref = refs
    i = pl.program_id(0)
    t = p0_ref[...] + p1_ref[...] - xs_ref[...]
    z = t * dinv_ref[...]
    h = jnp.dot(z, w_ref[...], preferred_element_type=jnp.float32) + b_ref[...]
    if residual:
        h = h + xres_ref[...]
    h_ref[...] = h

    @pl.when(i == 0)
    def _():
        acc_ref[0] = 0.0
        acc_ref[1] = 0.0

    acc_ref[0] += jnp.sum(h)
    acc_ref[1] += jnp.sum(h * h)

    @pl.when(i == NBLK - 1)
    def _():
        st_ref[0] = acc_ref[0]
        st_ref[1] = acc_ref[1]


def _make_mm(residual):
    row_spec = pl.BlockSpec((MB, D), lambda i: (i, 0))
    dinv_spec = pl.BlockSpec((MB, 1), lambda i: (i, 0))
    full_spec = pl.BlockSpec((D, D), lambda i: (0, 0))
    b_spec = pl.BlockSpec((1, D), lambda i: (0, 0))
    in_specs = [row_spec, row_spec, row_spec, dinv_spec, full_spec, b_spec]
    if residual:
        in_specs.append(row_spec)
    return pl.pallas_call(
        functools.partial(_mm_body, residual),
        grid=(NBLK,),
        in_specs=in_specs,
        out_specs=(
            row_spec,
            pl.BlockSpec(memory_space=pltpu.SMEM),
        ),
        out_shape=(
            jax.ShapeDtypeStruct((N, D), jnp.float32),
            jax.ShapeDtypeStruct((2,), jnp.float32),
        ),
        scratch_shapes=[pltpu.SMEM((2,), jnp.float32)],
    )


_mm0 = _make_mm(False)
_mm1 = _make_mm(True)


def _ln_body(scale_out, h_ref, st_ref, dinv_ref, w_ref, b_ref, out_ref):
    inv_n = 1.0 / (N * D)
    mean = st_ref[0] * inv_n
    var = st_ref[1] * inv_n - mean * mean
    rstd = lax.rsqrt(var + EPS)
    y = (h_ref[...] - mean) * rstd * w_ref[...] + b_ref[...]
    y = jnp.maximum(y, 0.0)
    if scale_out:
        y = y * dinv_ref[...]
    out_ref[...] = y


def _make_ln(scale_out):
    row_spec = pl.BlockSpec((MB, D), lambda i: (i, 0))
    dinv_spec = pl.BlockSpec((MB, 1), lambda i: (i, 0))
    b_spec = pl.BlockSpec((1, D), lambda i: (0, 0))
    return pl.pallas_call(
        functools.partial(_ln_body, scale_out),
        grid=(NBLK,),
        in_specs=[
            row_spec,
            pl.BlockSpec(memory_space=pltpu.SMEM),
            dinv_spec,
            b_spec,
            b_spec,
        ],
        out_specs=row_spec,
        out_shape=jax.ShapeDtypeStruct((N, D), jnp.float32),
    )


_ln0 = _make_ln(True)
_ln1 = _make_ln(False)


def kernel(x, edge_index, W0, b0, W1, b1, ln0_w, ln0_b, ln1_w, ln1_b):
    src = edge_index[0]
    dst = edge_index[1]
    ones = jnp.ones((N, DEGW), jnp.float32)
    b0r = b0.reshape(1, D)
    b1r = b1.reshape(1, D)
    ln0w = ln0_w.reshape(1, D)
    ln0b = ln0_b.reshape(1, D)
    ln1w = ln1_w.reshape(1, D)
    ln1b = ln1_b.reshape(1, D)

    degp = _deg_kernel(dst, ones)
    dinv, xs0 = _prep(degp[0], degp[1], x)

    p = _conv_kernel(xs0, src, dst)
    h0, st0 = _mm0(p[0], p[1], xs0, dinv, W0, b0r)
    xs1 = _ln0(h0, st0, dinv, ln0w, ln0b)

    q = _conv_kernel(xs1, src, dst)
    h1, st1 = _mm1(q[0], q[1], xs1, dinv, W1, b1r, x)
    out = _ln1(h1, st1, dinv, ln1w, ln1b)
    return out
